# baseline (speedup 1.0000x reference)
import jax
import jax.numpy as jnp
from jax.experimental import pallas as pl
from jax.experimental.pallas import tpu as pltpu

_BLKR = 80
_NBUF = 8


def _body(x_ref, w1_ref, b1_ref, w2_ref, b2_ref, a_ref, out_ref,
          sup_ref, sup2_ref, abuf, sem):
    n = x_ref.shape[0]
    blk = _BLKR
    nr = n // blk
    nsteps = 2 * nr

    def _fetch(block, slot):
        layer = block // nr
        r0 = (block % nr) * blk
        pltpu.make_async_copy(
            a_ref.at[layer, pl.ds(r0, blk), :], abuf.at[slot], sem.at[slot]
        ).start()

    def _wait(slot):
        pltpu.make_async_copy(
            a_ref.at[0, pl.ds(0, blk), :], abuf.at[slot], sem.at[slot]
        ).wait()

    for s in range(_NBUF):
        _fetch(s, s)
    sup_ref[...] = jnp.dot(
        x_ref[...], w1_ref[...], preferred_element_type=jnp.float32
    )

    def _step(i, carry):
        s = jax.lax.rem(i, _NBUF)
        _wait(s)

        @pl.when(i < nr)
        def _layer1():
            ht = jax.lax.dot_general(
                sup_ref[...], abuf[s], (((0,), (1,)), ((), ())),
                preferred_element_type=jnp.float32)
            h = ht.T
            h = h + b1_ref[...]
            h = jnp.where(h >= 0, h, 0.01 * h)
            sup2_ref[pl.ds(i * blk, blk), :] = jnp.dot(
                h, w2_ref[...], preferred_element_type=jnp.float32
            )

        @pl.when(i >= nr)
        def _layer2():
            h2t = jax.lax.dot_general(
                sup2_ref[...], abuf[s], (((0,), (1,)), ((), ())),
                preferred_element_type=jnp.float32)
            h2 = h2t.T
            h2 = h2 + b2_ref[...]
            m = jnp.max(h2, axis=1, keepdims=True)
            e = h2 - m
            lse = jnp.log(jnp.sum(jnp.exp(e), axis=1, keepdims=True))
            out_ref[pl.ds((i - nr) * blk, blk), :] = e - lse

        @pl.when(i + _NBUF < nsteps)
        def _refill():
            _fetch(i + _NBUF, s)

        return carry

    jax.lax.fori_loop(0, nsteps, _step, 0)


def kernel(x, A_s, W1, b1, W2, b2):
    n, fan_in = x.shape
    fan_mid = W1.shape[1]
    fan_out = W2.shape[1]
    b1r = b1.reshape(1, fan_mid)
    b2r = b2.reshape(1, fan_out)

    out = pl.pallas_call(
        _body,
        in_specs=[
            pl.BlockSpec((n, fan_in), lambda: (0, 0)),
            pl.BlockSpec((fan_in, fan_mid), lambda: (0, 0)),
            pl.BlockSpec((1, fan_mid), lambda: (0, 0)),
            pl.BlockSpec((fan_mid, fan_out), lambda: (0, 0)),
            pl.BlockSpec((1, fan_out), lambda: (0, 0)),
            pl.BlockSpec(memory_space=pl.ANY),
        ],
        out_specs=pl.BlockSpec((n, fan_out), lambda: (0, 0)),
        out_shape=jax.ShapeDtypeStruct((n, fan_out), jnp.float32),
        scratch_shapes=[
            pltpu.VMEM((n, fan_mid), jnp.float32),
            pltpu.VMEM((n, fan_out), jnp.float32),
            pltpu.VMEM((_NBUF, _BLKR, n), jnp.float32),
            pltpu.SemaphoreType.DMA((_NBUF,)),
        ],
    )(x, W1, b1r, W2, b2r, A_s)

    return out


# speedup vs baseline: 1.2344x; 1.2344x over previous
import jax
import jax.numpy as jnp
from jax.experimental import pallas as pl
from jax.experimental.pallas import tpu as pltpu

_BLKR = 200
_NBUF = 4


def _body(x_ref, w1_ref, b1_ref, w2_ref, b2_ref, a_ref, out_ref,
          sup_ref, sup2_ref, abuf, sem):
    n = x_ref.shape[0]
    blk = _BLKR
    nr = n // blk
    nsteps = 2 * nr

    def _fetch(block, slot):
        layer = block // nr
        r0 = (block % nr) * blk
        pltpu.make_async_copy(
            a_ref.at[layer, pl.ds(r0, blk), :], abuf.at[slot], sem.at[slot]
        ).start()

    def _wait(slot):
        pltpu.make_async_copy(
            a_ref.at[0, pl.ds(0, blk), :], abuf.at[slot], sem.at[slot]
        ).wait()

    for s in range(_NBUF):
        _fetch(s, s)
    sup_ref[...] = jnp.dot(
        x_ref[...], w1_ref[...], preferred_element_type=jnp.float32
    )

    def _step(i, carry):
        s = jax.lax.rem(i, _NBUF)
        _wait(s)

        @pl.when(i < nr)
        def _layer1():
            ht = jax.lax.dot_general(
                sup_ref[...], abuf[s], (((0,), (1,)), ((), ())),
                preferred_element_type=jnp.float32)
            ht = ht + b1_ref[...]
            ht = jnp.where(ht >= 0, ht, 0.01 * ht)
            s2t = jax.lax.dot_general(
                w2_ref[...], ht, (((0,), (0,)), ((), ())),
                preferred_element_type=jnp.float32)
            sup2_ref[pl.ds(i * blk, blk), :] = s2t.T

        @pl.when(i >= nr)
        def _layer2():
            h2t = jax.lax.dot_general(
                sup2_ref[...], abuf[s], (((0,), (1,)), ((), ())),
                preferred_element_type=jnp.float32)
            h2t = h2t + b2_ref[...]
            m = jnp.max(h2t, axis=0, keepdims=True)
            e = h2t - m
            lse = jnp.log(jnp.sum(jnp.exp(e), axis=0, keepdims=True))
            out_ref[pl.ds((i - nr) * blk, blk), :] = (e - lse).T

        @pl.when(i + _NBUF < nsteps)
        def _refill():
            _fetch(i + _NBUF, s)

        return carry

    jax.lax.fori_loop(0, nsteps, _step, 0)


def kernel(x, A_s, W1, b1, W2, b2):
    n, fan_in = x.shape
    fan_mid = W1.shape[1]
    fan_out = W2.shape[1]
    b1r = b1.reshape(fan_mid, 1)
    b2r = b2.reshape(fan_out, 1)

    out = pl.pallas_call(
        _body,
        in_specs=[
            pl.BlockSpec((n, fan_in), lambda: (0, 0)),
            pl.BlockSpec((fan_in, fan_mid), lambda: (0, 0)),
            pl.BlockSpec((fan_mid, 1), lambda: (0, 0)),
            pl.BlockSpec((fan_mid, fan_out), lambda: (0, 0)),
            pl.BlockSpec((fan_out, 1), lambda: (0, 0)),
            pl.BlockSpec(memory_space=pl.ANY),
        ],
        out_specs=pl.BlockSpec((n, fan_out), lambda: (0, 0)),
        out_shape=jax.ShapeDtypeStruct((n, fan_out), jnp.float32),
        scratch_shapes=[
            pltpu.VMEM((n, fan_mid), jnp.float32),
            pltpu.VMEM((n, fan_out), jnp.float32),
            pltpu.VMEM((_NBUF, _BLKR, n), jnp.float32),
            pltpu.SemaphoreType.DMA((_NBUF,)),
        ],
    )(x, W1, b1r, W2, b2r, A_s)

    return out


# speedup vs baseline: 1.2479x; 1.0109x over previous
import jax
import jax.numpy as jnp
from jax.experimental import pallas as pl
from jax.experimental.pallas import tpu as pltpu

_BLK = 400


def _body(x_ref, w1_ref, b1_ref, w2_ref, b2_ref, a_ref, out_ref,
          sup_ref, sup2_ref):
    i = pl.program_id(0)
    nr = pl.num_programs(0) // 2
    blk = a_ref.shape[1]

    @pl.when(i == 0)
    def _init():
        sup_ref[...] = jnp.dot(
            x_ref[...], w1_ref[...], preferred_element_type=jnp.float32
        )

    @pl.when(i < nr)
    def _layer1():
        ht = jax.lax.dot_general(
            sup_ref[...], a_ref[0], (((0,), (1,)), ((), ())),
            preferred_element_type=jnp.float32)
        h = ht.T
        h = h + b1_ref[...]
        h = jnp.where(h >= 0, h, 0.01 * h)
        sup2_ref[pl.ds(i * blk, blk), :] = jnp.dot(
            h, w2_ref[...], preferred_element_type=jnp.float32
        )

    @pl.when(i >= nr)
    def _layer2():
        h2t = jax.lax.dot_general(
            sup2_ref[...], a_ref[0], (((0,), (1,)), ((), ())),
            preferred_element_type=jnp.float32)
        h2 = h2t.T
        h2 = h2 + b2_ref[...]
        m = jnp.max(h2, axis=1, keepdims=True)
        e = h2 - m
        lse = jnp.log(jnp.sum(jnp.exp(e), axis=1, keepdims=True))
        out_ref[...] = e - lse


def kernel(x, A_s, W1, b1, W2, b2):
    n, fan_in = x.shape
    fan_mid = W1.shape[1]
    fan_out = W2.shape[1]
    blk = _BLK
    nr = n // blk
    b1r = b1.reshape(1, fan_mid)
    b2r = b2.reshape(1, fan_out)

    out = pl.pallas_call(
        _body,
        grid=(2 * nr,),
        in_specs=[
            pl.BlockSpec((n, fan_in), lambda i: (0, 0)),
            pl.BlockSpec((fan_in, fan_mid), lambda i: (0, 0)),
            pl.BlockSpec((1, fan_mid), lambda i: (0, 0)),
            pl.BlockSpec((fan_mid, fan_out), lambda i: (0, 0)),
            pl.BlockSpec((1, fan_out), lambda i: (0, 0)),
            pl.BlockSpec((1, blk, n), lambda i: (i // nr, i % nr, 0)),
        ],
        out_specs=pl.BlockSpec(
            (blk, fan_out), lambda i: (jnp.maximum(i - nr, 0), 0)
        ),
        out_shape=jax.ShapeDtypeStruct((n, fan_out), jnp.float32),
        scratch_shapes=[
            pltpu.VMEM((n, fan_mid), jnp.float32),
            pltpu.VMEM((n, fan_out), jnp.float32),
        ],
        compiler_params=pltpu.CompilerParams(
            dimension_semantics=("arbitrary",),
        ),
    )(x, W1, b1r, W2, b2r, A_s)

    return out
